# Initial kernel scaffold; baseline (speedup 1.0000x reference)
import jax
import jax.numpy as jnp
from jax.experimental import pallas as pl


def kernel(edge_index, node_kind_ids, edge_attr, emb, msg_w1, msg_b1, msg_w2, msg_b2, gru_wih, gru_bih, gru_whh, gru_bhh, eh_w1, eh_b1, eh_w2, eh_b2):
    raise NotImplementedError("write your pallas kernel here")


# speedup vs baseline: 1.1212x; 1.1212x over previous
import functools

import jax
import jax.numpy as jnp
from jax import lax
from jax.experimental import pallas as pl
from jax.experimental.pallas import tpu as pltpu
from jax.experimental.pallas import tpu_sc as plsc

N_NODES = 50000
N_EDGES = 800000
D = 64
NC = 2
NS = 16
NW = NC * NS
NP = 53248
EP = 819200
HALF = NP // 2
ROWS_PER_TILE = HALF // NS
ZCH = ROWS_PER_TILE // 8
SCH = 256
GCH = 1024
PREC = jax.lax.Precision.HIGHEST

_mesh = lambda: plsc.VectorSubcoreMesh(core_axis_name="c", subcore_axis_name="s")


def _sc_gather(table, idxs, chunk):
    n_idx = len(idxs)
    B = idxs[0].shape[0]
    d = table.shape[1]
    per_w = B // NW
    n_chunks = per_w // chunk
    R = chunk // 128

    def body(*refs):
        table_r = refs[0]
        idx_rs = refs[1:1 + n_idx]
        out_rs = refs[1 + n_idx:1 + 2 * n_idx]
        idx_v, rows_v, sem = refs[1 + 2 * n_idx:]
        wid = lax.axis_index("s") * NC + lax.axis_index("c")
        base_w = wid * per_w

        def chunk_body(i, carry):
            base = base_w + i * chunk
            for t in range(n_idx):
                for j in range(R):
                    pltpu.sync_copy(idx_rs[t].at[pl.ds(base + j * 128, 128)],
                                    idx_v.at[j])
                cps = [pltpu.async_copy(table_r.at[idx_v.at[j]],
                                        rows_v.at[pl.ds(j * 128, 128)], sem)
                       for j in range(R)]
                for cp in cps:
                    cp.wait()
                pltpu.sync_copy(rows_v, out_rs[t].at[pl.ds(base, chunk)])
            return carry

        lax.fori_loop(0, n_chunks, chunk_body, 0)

    f = pl.kernel(
        body,
        out_type=tuple(jax.ShapeDtypeStruct((B, d), jnp.float32)
                       for _ in range(n_idx)),
        mesh=_mesh(),
        compiler_params=pltpu.CompilerParams(use_tc_tiling_on_sc=False),
        scratch_types=[
            pltpu.VMEM((R, 128), jnp.int32),
            pltpu.VMEM((chunk, d), jnp.float32),
            pltpu.SemaphoreType.DMA,
        ],
    )
    out = f(table, *idxs)
    return (out,) if n_idx == 1 and not isinstance(out, (tuple, list)) else tuple(out)


def _sc_scatter_add(m, dst, zeros):
    def body(m_r, dst_r, zero_r, agg_r, m_v, raw_v, idx_v, agg_s):
        cid = lax.axis_index("c")
        sid = lax.axis_index("s")
        base_node = cid * HALF
        stripe = sid * ROWS_PER_TILE
        z_v = m_v.at[pl.ds(0, ZCH)]
        pltpu.sync_copy(zero_r, z_v)
        for k in range(8):
            pltpu.sync_copy(z_v, agg_s.at[pl.ds(stripe + k * ZCH, ZCH)])
        plsc.subcore_barrier()

        def chunk_body(i, carry):
            base = (i * NS + sid) * SCH
            pltpu.sync_copy(m_r.at[pl.ds(base, SCH)], m_v)
            pltpu.sync_copy(dst_r.at[pl.ds(base, SCH)], raw_v)
            for g in range(SCH // 16):
                v = raw_v[pl.ds(g * 16, 16)]
                lo = v - base_node
                inb = (lo >= 0) & (lo < HALF)
                idx_v[g // 8, pl.ds((g % 8) * 16, 16)] = jnp.where(inb, lo, HALF)
            for j in range(SCH // 128):
                pltpu.sync_copy(m_v.at[pl.ds(j * 128, 128)],
                                agg_s.at[idx_v.at[j]], add=True)
            return carry

        lax.fori_loop(0, (EP // SCH) // NS, chunk_body, 0)
        plsc.subcore_barrier()
        for k in range(8):
            pltpu.sync_copy(agg_s.at[pl.ds(stripe + k * ZCH, ZCH)], z_v)
            pltpu.sync_copy(z_v, agg_r.at[pl.ds(base_node + stripe + k * ZCH, ZCH)])

    f = pl.kernel(
        body,
        out_type=jax.ShapeDtypeStruct((NP, D), jnp.float32),
        mesh=_mesh(),
        compiler_params=pltpu.CompilerParams(use_tc_tiling_on_sc=False),
        scratch_types=[
            pltpu.VMEM((SCH, D), jnp.float32),
            pltpu.VMEM((SCH,), jnp.int32),
            pltpu.VMEM((SCH // 128, 128), jnp.int32),
            pltpu.VMEM_SHARED((HALF + 8, D), jnp.float32),
        ],
    )
    return f(m, dst, zeros)


def _tc_edge_mlp(hs, hd, ea, w1sT, w1dT, w1eT, b1, w2T, b2):
    B = 2048
    grid = EP // B

    def body(hs_r, hd_r, ea_r, w1s_r, w1d_r, w1e_r, b1_r, w2_r, b2_r, o_r):
        t = jnp.dot(hs_r[...], w1s_r[...], preferred_element_type=jnp.float32,
                    precision=PREC)
        t += jnp.dot(hd_r[...], w1d_r[...], preferred_element_type=jnp.float32,
                     precision=PREC)
        t += jnp.dot(ea_r[...], w1e_r[...], preferred_element_type=jnp.float32,
                     precision=PREC)
        t += b1_r[...]
        t = t * jax.nn.sigmoid(t)
        o_r[...] = jnp.dot(t, w2_r[...], preferred_element_type=jnp.float32,
                           precision=PREC) + b2_r[...]

    full = lambda shape: pl.BlockSpec(shape, lambda i: (0, 0))
    return pl.pallas_call(
        body,
        grid=(grid,),
        in_specs=[
            pl.BlockSpec((B, D), lambda i: (i, 0)),
            pl.BlockSpec((B, D), lambda i: (i, 0)),
            pl.BlockSpec((B, 4), lambda i: (i, 0)),
            full((D, D)), full((D, D)), full((4, D)), full((1, D)),
            full((D, D)), full((1, D)),
        ],
        out_specs=pl.BlockSpec((B, D), lambda i: (i, 0)),
        out_shape=jax.ShapeDtypeStruct((EP, D), jnp.float32),
    )(hs, hd, ea, w1sT, w1dT, w1eT, b1, w2T, b2)


def _tc_gru(h, agg, wih, bih, whh, bhh):
    B = 2048
    grid = NP // B

    def body(h_r, agg_r, wr_r, wz_r, wn_r, ur_r, uz_r, un_r,
             br_r, bz_r, bn_r, cr_r, cz_r, cn_r, o_r):
        h_v = h_r[...]
        a_v = agg_r[...]
        dot = lambda x, w: jnp.dot(x, w, preferred_element_type=jnp.float32,
                                   precision=PREC)
        r = jax.nn.sigmoid(dot(a_v, wr_r[...]) + br_r[...]
                           + dot(h_v, ur_r[...]) + cr_r[...])
        z = jax.nn.sigmoid(dot(a_v, wz_r[...]) + bz_r[...]
                           + dot(h_v, uz_r[...]) + cz_r[...])
        nn_ = jnp.tanh(dot(a_v, wn_r[...]) + bn_r[...]
                       + r * (dot(h_v, un_r[...]) + cn_r[...]))
        o_r[...] = (1.0 - z) * nn_ + z * h_v

    wihT = wih.T
    whhT = whh.T
    ws = [wihT[:, i * D:(i + 1) * D] for i in range(3)]
    us = [whhT[:, i * D:(i + 1) * D] for i in range(3)]
    bs = [bih.reshape(3, D)[i].reshape(1, D) for i in range(3)]
    cs = [bhh.reshape(3, D)[i].reshape(1, D) for i in range(3)]
    full = lambda shape: pl.BlockSpec(shape, lambda i: (0, 0))
    return pl.pallas_call(
        body,
        grid=(grid,),
        in_specs=[pl.BlockSpec((B, D), lambda i: (i, 0)),
                  pl.BlockSpec((B, D), lambda i: (i, 0))]
        + [full((D, D))] * 6 + [full((1, D))] * 6,
        out_specs=pl.BlockSpec((B, D), lambda i: (i, 0)),
        out_shape=jax.ShapeDtypeStruct((NP, D), jnp.float32),
    )(h, agg, *ws, *us, *bs, *cs)


def _tc_head(hs, hd, ea, w1sT, w1dT, w1eT, b1, w2row, b2):
    B = 2048
    grid = EP // B

    def body(hs_r, hd_r, ea_r, w1s_r, w1d_r, w1e_r, b1_r, w2_r, b2_r, o_r):
        t = jnp.dot(hs_r[...], w1s_r[...], preferred_element_type=jnp.float32,
                    precision=PREC)
        t += jnp.dot(hd_r[...], w1d_r[...], preferred_element_type=jnp.float32,
                     precision=PREC)
        t += jnp.dot(ea_r[...], w1e_r[...], preferred_element_type=jnp.float32,
                     precision=PREC)
        t += b1_r[...]
        t = t * jax.nn.sigmoid(t)
        lg = jnp.sum(t * w2_r[...], axis=1, keepdims=True) + b2_r[...]
        o_r[...] = jax.nn.sigmoid(lg)

    full = lambda shape: pl.BlockSpec(shape, lambda i: (0, 0))
    return pl.pallas_call(
        body,
        grid=(grid,),
        in_specs=[
            pl.BlockSpec((B, D), lambda i: (i, 0)),
            pl.BlockSpec((B, D), lambda i: (i, 0)),
            pl.BlockSpec((B, 4), lambda i: (i, 0)),
            full((D, D)), full((D, D)), full((4, D)), full((1, D)),
            full((1, D)), full((1, 1)),
        ],
        out_specs=pl.BlockSpec((B, 1), lambda i: (i, 0)),
        out_shape=jax.ShapeDtypeStruct((EP, 1), jnp.float32),
    )(hs, hd, ea, w1sT, w1dT, w1eT, b1, w2row, b2)


def kernel(edge_index, node_kind_ids, edge_attr, emb, msg_w1, msg_b1, msg_w2,
           msg_b2, gru_wih, gru_bih, gru_whh, gru_bhh, eh_w1, eh_b1, eh_w2,
           eh_b2):
    src = edge_index[0].astype(jnp.int32)
    dst = edge_index[1].astype(jnp.int32)
    src_p = jnp.pad(src, (0, EP - N_EDGES))
    dst_p = jnp.pad(dst, (0, EP - N_EDGES), constant_values=1 << 28)
    ea_p = jnp.pad(edge_attr, ((0, EP - N_EDGES), (0, 0)))
    ids_p = jnp.pad(node_kind_ids.astype(jnp.int32), (0, NP - N_NODES))
    zeros = jnp.zeros((ZCH, D), jnp.float32)

    (h,) = _sc_gather(emb, (ids_p,), NP // NW)

    for l in range(3):
        w1 = msg_w1[l]
        w1sT = w1[:, :D].T
        w1dT = w1[:, D:2 * D].T
        w1eT = w1[:, 2 * D:].T
        b1 = msg_b1[l].reshape(1, D)
        w2T = msg_w2[l].T
        b2 = msg_b2[l].reshape(1, D)

        hs, hd = _sc_gather(h, (src_p, dst_p), GCH)
        m = _tc_edge_mlp(hs, hd, ea_p, w1sT, w1dT, w1eT, b1, w2T, b2)
        agg = _sc_scatter_add(m, dst_p, zeros)
        h = _tc_gru(h, agg, gru_wih[l], gru_bih[l], gru_whh[l], gru_bhh[l])

    hs, hd = _sc_gather(h, (src_p, dst_p), GCH)
    out = _tc_head(hs, hd, ea_p, eh_w1[:, :D].T, eh_w1[:, D:2 * D].T,
                   eh_w1[:, 2 * D:].T, eh_b1.reshape(1, D),
                   eh_w2.reshape(1, D), eh_b2.reshape(1, 1))
    return out[:N_EDGES, 0]


# speedup vs baseline: 1.5443x; 1.3774x over previous
import functools

import jax
import jax.numpy as jnp
from jax import lax
from jax.experimental import pallas as pl
from jax.experimental.pallas import tpu as pltpu
from jax.experimental.pallas import tpu_sc as plsc

N_NODES = 50000
N_EDGES = 800000
D = 64
NC = 2
NS = 16
NW = NC * NS
NP = 53248
EP = 819200
HALF = NP // 2
ROWS_PER_TILE = HALF // NS
ZCH = ROWS_PER_TILE // 16
SCH = 128
GCH = 640
F32 = jnp.float32
BF16 = jnp.bfloat16
HIGHEST = jax.lax.Precision.HIGHEST

_mesh = lambda: plsc.VectorSubcoreMesh(core_axis_name="c", subcore_axis_name="s")
_sc_params = lambda: pltpu.CompilerParams(use_tc_tiling_on_sc=False)


def _sc_gather(table, idxs, chunk):
    n_idx = len(idxs)
    B = idxs[0].shape[0] * 128
    d = table.shape[1]
    per_w = B // NW
    n_chunks = per_w // chunk
    R = chunk // 128
    PW = per_w // 128

    def body(*refs):
        table_r = refs[0]
        idx_rs = refs[1:1 + n_idx]
        out_rs = refs[1 + n_idx:1 + 2 * n_idx]
        scratch = refs[1 + 2 * n_idx:]
        idx_v = scratch[0:n_idx]
        rows_v = scratch[n_idx:2 * n_idx]
        sem_g = scratch[2 * n_idx]
        sem_w = scratch[2 * n_idx + 1:2 * n_idx + 1 + n_idx]
        wid = lax.axis_index("s") * NC + lax.axis_index("c")
        base_w = wid * per_w
        row_w = wid * PW

        def chunk_body(i, carry):
            base = base_w + i * chunk
            for t in range(n_idx):
                pltpu.sync_copy(idx_rs[t].at[pl.ds(row_w + i * R, R)], idx_v[t])

                @pl.when(i > 0)
                def _():
                    pltpu.make_async_copy(
                        rows_v[t], out_rs[t].at[pl.ds(base, chunk)],
                        sem_w[t]).wait()

                cps = [pltpu.async_copy(table_r.at[idx_v[t].at[j]],
                                        rows_v[t].at[pl.ds(j * 128, 128)],
                                        sem_g)
                       for j in range(R)]
                for cp in cps:
                    cp.wait()
                pltpu.async_copy(rows_v[t], out_rs[t].at[pl.ds(base, chunk)],
                                 sem_w[t])
            return carry

        lax.fori_loop(0, n_chunks, chunk_body, 0)
        last = base_w + (n_chunks - 1) * chunk
        for t in range(n_idx):
            pltpu.make_async_copy(rows_v[t], out_rs[t].at[pl.ds(last, chunk)],
                                  sem_w[t]).wait()

    f = pl.kernel(
        body,
        out_type=tuple(jax.ShapeDtypeStruct((B, d), F32)
                       for _ in range(n_idx)),
        mesh=_mesh(),
        compiler_params=_sc_params(),
        scratch_types=[pltpu.VMEM((R, 128), jnp.int32) for _ in range(n_idx)]
        + [pltpu.VMEM((chunk, d), F32) for _ in range(n_idx)]
        + [pltpu.SemaphoreType.DMA] * (1 + n_idx),
    )
    out = f(table, *idxs)
    return (out,) if n_idx == 1 and not isinstance(out, (tuple, list)) else tuple(out)


def _sc_scatter_add(m, dst2d, zeros):
    n_per_tile = (EP // SCH) // NS
    n_pairs = n_per_tile // 2

    def body(m_r, dst_r, zero_r, agg_r, m_v0, m_v1, raw_v0, raw_v1,
             idx_v0, idx_v1, sem0, sem1, agg_s):
        cid = lax.axis_index("c")
        sid = lax.axis_index("s")
        base_node = cid * HALF
        stripe = sid * ROWS_PER_TILE
        m_v = (m_v0, m_v1)
        raw_v = (raw_v0, raw_v1)
        idx_v = (idx_v0, idx_v1)
        sem = (sem0, sem1)
        z_v = m_v0.at[pl.ds(0, ZCH)]

        pltpu.sync_copy(zero_r, z_v)
        for k in range(16):
            pltpu.sync_copy(z_v, agg_s.at[pl.ds(stripe + k * ZCH, ZCH)])
        plsc.subcore_barrier()

        def start_loads(u, b):
            base = (u * NS + sid) * SCH
            pltpu.async_copy(m_r.at[pl.ds(base, SCH)], m_v[b], sem[b])
            pltpu.async_copy(dst_r.at[pl.ds(base // 128, 1)], raw_v[b], sem[b])

        def finish_loads(u, b):
            base = (u * NS + sid) * SCH
            pltpu.make_async_copy(m_r.at[pl.ds(base, SCH)], m_v[b],
                                  sem[b]).wait()
            pltpu.make_async_copy(dst_r.at[pl.ds(base // 128, 1)], raw_v[b],
                                  sem[b]).wait()

        start_loads(0, 0)
        start_loads(1, 1)

        def pair_body(p, carry):
            for b in range(2):
                u = 2 * p + b
                finish_loads(u, b)
                for g in range(SCH // 16):
                    v = raw_v[b][0, pl.ds(g * 16, 16)]
                    lo = v - base_node
                    inb = (lo >= 0) & (lo < HALF)
                    idx_v[b][0, pl.ds(g * 16, 16)] = jnp.where(inb, lo, HALF)
                pltpu.sync_copy(m_v[b], agg_s.at[idx_v[b].at[0]], add=True)

                @pl.when(u + 2 < n_per_tile)
                def _():
                    start_loads(u + 2, b)
            return carry

        lax.fori_loop(0, n_pairs, pair_body, 0)
        plsc.subcore_barrier()
        for k in range(16):
            pltpu.sync_copy(agg_s.at[pl.ds(stripe + k * ZCH, ZCH)], z_v)
            pltpu.sync_copy(z_v, agg_r.at[pl.ds(base_node + stripe + k * ZCH, ZCH)])

    f = pl.kernel(
        body,
        out_type=jax.ShapeDtypeStruct((NP, D), F32),
        mesh=_mesh(),
        compiler_params=_sc_params(),
        scratch_types=[
            pltpu.VMEM((SCH, D), F32), pltpu.VMEM((SCH, D), F32),
            pltpu.VMEM((1, SCH), jnp.int32), pltpu.VMEM((1, SCH), jnp.int32),
            pltpu.VMEM((1, SCH), jnp.int32), pltpu.VMEM((1, SCH), jnp.int32),
            pltpu.SemaphoreType.DMA, pltpu.SemaphoreType.DMA,
            pltpu.VMEM_SHARED((HALF + 8, D), F32),
        ],
    )
    return f(m, dst2d, zeros)


def _bdot(x, w):
    return jnp.dot(x.astype(BF16), w.astype(BF16),
                   preferred_element_type=F32)


def _tc_edge_mlp(hs, hd, ea, w1sT, w1dT, w1eT, b1, w2T, b2):
    B = 2048
    grid = EP // B

    def body(hs_r, hd_r, ea_r, w1s_r, w1d_r, w1e_r, b1_r, w2_r, b2_r, o_r):
        t = _bdot(hs_r[...], w1s_r[...])
        t += _bdot(hd_r[...], w1d_r[...])
        t += _bdot(ea_r[...], w1e_r[...])
        t += b1_r[...]
        t = t * jax.nn.sigmoid(t)
        o_r[...] = _bdot(t, w2_r[...]) + b2_r[...]

    full = lambda shape: pl.BlockSpec(shape, lambda i: (0, 0))
    return pl.pallas_call(
        body,
        grid=(grid,),
        in_specs=[
            pl.BlockSpec((B, D), lambda i: (i, 0)),
            pl.BlockSpec((B, D), lambda i: (i, 0)),
            pl.BlockSpec((B, 4), lambda i: (i, 0)),
            full((D, D)), full((D, D)), full((4, D)), full((1, D)),
            full((D, D)), full((1, D)),
        ],
        out_specs=pl.BlockSpec((B, D), lambda i: (i, 0)),
        out_shape=jax.ShapeDtypeStruct((EP, D), F32),
    )(hs, hd, ea, w1sT, w1dT, w1eT, b1, w2T, b2)


def _tc_gru(h, agg, wih, bih, whh, bhh):
    B = 2048
    grid = NP // B

    def body(h_r, agg_r, wr_r, wz_r, wn_r, ur_r, uz_r, un_r,
             br_r, bz_r, bn_r, cr_r, cz_r, cn_r, o_r):
        h_v = h_r[...]
        a_v = agg_r[...]
        dot = lambda x, w: jnp.dot(x, w, preferred_element_type=F32,
                                   precision=HIGHEST)
        r = jax.nn.sigmoid(dot(a_v, wr_r[...]) + br_r[...]
                           + dot(h_v, ur_r[...]) + cr_r[...])
        z = jax.nn.sigmoid(dot(a_v, wz_r[...]) + bz_r[...]
                           + dot(h_v, uz_r[...]) + cz_r[...])
        nn_ = jnp.tanh(dot(a_v, wn_r[...]) + bn_r[...]
                       + r * (dot(h_v, un_r[...]) + cn_r[...]))
        o_r[...] = (1.0 - z) * nn_ + z * h_v

    wihT = wih.T
    whhT = whh.T
    ws = [wihT[:, i * D:(i + 1) * D] for i in range(3)]
    us = [whhT[:, i * D:(i + 1) * D] for i in range(3)]
    bs = [bih.reshape(3, D)[i].reshape(1, D) for i in range(3)]
    cs = [bhh.reshape(3, D)[i].reshape(1, D) for i in range(3)]
    full = lambda shape: pl.BlockSpec(shape, lambda i: (0, 0))
    return pl.pallas_call(
        body,
        grid=(grid,),
        in_specs=[pl.BlockSpec((B, D), lambda i: (i, 0)),
                  pl.BlockSpec((B, D), lambda i: (i, 0))]
        + [full((D, D))] * 6 + [full((1, D))] * 6,
        out_specs=pl.BlockSpec((B, D), lambda i: (i, 0)),
        out_shape=jax.ShapeDtypeStruct((NP, D), F32),
    )(h, agg, *ws, *us, *bs, *cs)


def _tc_head(hs, hd, ea, w1sT, w1dT, w1eT, b1, w2row, b2):
    B = 2048
    grid = EP // B

    def body(hs_r, hd_r, ea_r, w1s_r, w1d_r, w1e_r, b1_r, w2_r, b2_r, o_r):
        t = _bdot(hs_r[...], w1s_r[...])
        t += _bdot(hd_r[...], w1d_r[...])
        t += _bdot(ea_r[...], w1e_r[...])
        t += b1_r[...]
        t = t * jax.nn.sigmoid(t)
        lg = jnp.sum(t * w2_r[...], axis=1, keepdims=True) + b2_r[...]
        o_r[...] = jax.nn.sigmoid(lg)

    full = lambda shape: pl.BlockSpec(shape, lambda i: (0, 0))
    return pl.pallas_call(
        body,
        grid=(grid,),
        in_specs=[
            pl.BlockSpec((B, D), lambda i: (i, 0)),
            pl.BlockSpec((B, D), lambda i: (i, 0)),
            pl.BlockSpec((B, 4), lambda i: (i, 0)),
            full((D, D)), full((D, D)), full((4, D)), full((1, D)),
            full((1, D)), full((1, 1)),
        ],
        out_specs=pl.BlockSpec((B, 1), lambda i: (i, 0)),
        out_shape=jax.ShapeDtypeStruct((EP, 1), F32),
    )(hs, hd, ea, w1sT, w1dT, w1eT, b1, w2row, b2)


def kernel(edge_index, node_kind_ids, edge_attr, emb, msg_w1, msg_b1, msg_w2,
           msg_b2, gru_wih, gru_bih, gru_whh, gru_bhh, eh_w1, eh_b1, eh_w2,
           eh_b2):
    src = edge_index[0].astype(jnp.int32)
    dst = edge_index[1].astype(jnp.int32)
    src2 = jnp.pad(src, (0, EP - N_EDGES)).reshape(EP // 128, 128)
    dst2 = jnp.pad(dst, (0, EP - N_EDGES),
                   constant_values=1 << 28).reshape(EP // 128, 128)
    ea_p = jnp.pad(edge_attr, ((0, EP - N_EDGES), (0, 0)))
    ids2 = jnp.pad(node_kind_ids.astype(jnp.int32),
                   (0, NP - N_NODES)).reshape(NP // 128, 128)
    zeros = jnp.zeros((ZCH, D), F32)

    (h,) = _sc_gather(emb, (ids2,), 128)

    for l in range(3):
        w1 = msg_w1[l]
        w1sT = w1[:, :D].T
        w1dT = w1[:, D:2 * D].T
        w1eT = w1[:, 2 * D:].T
        b1 = msg_b1[l].reshape(1, D)
        w2T = msg_w2[l].T
        b2 = msg_b2[l].reshape(1, D)

        hs, hd = _sc_gather(h, (src2, dst2), GCH)
        m = _tc_edge_mlp(hs, hd, ea_p, w1sT, w1dT, w1eT, b1, w2T, b2)
        agg = _sc_scatter_add(m, dst2, zeros)
        h = _tc_gru(h, agg, gru_wih[l], gru_bih[l], gru_whh[l], gru_bhh[l])

    hs, hd = _sc_gather(h, (src2, dst2), GCH)
    out = _tc_head(hs, hd, ea_p, eh_w1[:, :D].T, eh_w1[:, D:2 * D].T,
                   eh_w1[:, 2 * D:].T, eh_b1.reshape(1, D),
                   eh_w2.reshape(1, D), eh_b2.reshape(1, 1))
    return out[:N_EDGES, 0]


# speedup vs baseline: 1.7121x; 1.1086x over previous
import functools

import jax
import jax.numpy as jnp
from jax import lax
from jax.experimental import pallas as pl
from jax.experimental.pallas import tpu as pltpu
from jax.experimental.pallas import tpu_sc as plsc

N_NODES = 50000
N_EDGES = 800000
D = 64
NC = 2
NS = 16
NW = NC * NS
NP = 53248
EP = 819200
HALF = NP // 2
ROWS_PER_TILE = HALF // NS
ZCH = ROWS_PER_TILE // 16
SCH = 128
GCH = 640
F32 = jnp.float32
BF16 = jnp.bfloat16
HIGHEST = jax.lax.Precision.HIGHEST

_mesh = lambda: plsc.VectorSubcoreMesh(core_axis_name="c", subcore_axis_name="s")
_sc_params = lambda: pltpu.CompilerParams(use_tc_tiling_on_sc=False)


def _sc_gather(table, idxs, chunk):
    n_idx = len(idxs)
    B = idxs[0].shape[0] * 128
    d = table.shape[1]
    dt = table.dtype
    per_w = B // NW
    n_chunks = per_w // chunk
    R = chunk // 128
    PW = per_w // 128

    def body(*refs):
        table_r = refs[0]
        idx_rs = refs[1:1 + n_idx]
        out_rs = refs[1 + n_idx:1 + 2 * n_idx]
        scratch = refs[1 + 2 * n_idx:]
        idx_v = scratch[0:n_idx]
        rows_v = scratch[n_idx:2 * n_idx]
        sem_g = scratch[2 * n_idx]
        sem_w = scratch[2 * n_idx + 1:2 * n_idx + 1 + n_idx]
        wid = lax.axis_index("s") * NC + lax.axis_index("c")
        base_w = wid * per_w
        row_w = wid * PW

        def chunk_body(i, carry):
            base = base_w + i * chunk
            for t in range(n_idx):
                pltpu.sync_copy(idx_rs[t].at[pl.ds(row_w + i * R, R)], idx_v[t])

                @pl.when(i > 0)
                def _():
                    pltpu.make_async_copy(
                        rows_v[t], out_rs[t].at[pl.ds(base, chunk)],
                        sem_w[t]).wait()

                cps = [pltpu.async_copy(table_r.at[idx_v[t].at[j]],
                                        rows_v[t].at[pl.ds(j * 128, 128)],
                                        sem_g)
                       for j in range(R)]
                for cp in cps:
                    cp.wait()
                pltpu.async_copy(rows_v[t], out_rs[t].at[pl.ds(base, chunk)],
                                 sem_w[t])
            return carry

        lax.fori_loop(0, n_chunks, chunk_body, 0)
        last = base_w + (n_chunks - 1) * chunk
        for t in range(n_idx):
            pltpu.make_async_copy(rows_v[t], out_rs[t].at[pl.ds(last, chunk)],
                                  sem_w[t]).wait()

    f = pl.kernel(
        body,
        out_type=tuple(jax.ShapeDtypeStruct((B, d), dt)
                       for _ in range(n_idx)),
        mesh=_mesh(),
        compiler_params=_sc_params(),
        scratch_types=[pltpu.VMEM((R, 128), jnp.int32) for _ in range(n_idx)]
        + [pltpu.VMEM((chunk, d), dt) for _ in range(n_idx)]
        + [pltpu.SemaphoreType.DMA] * (1 + n_idx),
    )
    out = f(table, *idxs)
    return (out,) if n_idx == 1 and not isinstance(out, (tuple, list)) else tuple(out)


def _sc_scatter_add(m, dst2d, zeros):
    n_per_tile = (EP // SCH) // NS
    n_pairs = n_per_tile // 2

    def body(m_r, dst_r, zero_r, agg_r, m_v0, m_v1, raw_v0, raw_v1,
             idx_v0, idx_v1, sem0, sem1, agg_s):
        cid = lax.axis_index("c")
        sid = lax.axis_index("s")
        base_node = cid * HALF
        stripe = sid * ROWS_PER_TILE
        m_v = (m_v0, m_v1)
        raw_v = (raw_v0, raw_v1)
        idx_v = (idx_v0, idx_v1)
        sem = (sem0, sem1)
        z_v = m_v0.at[pl.ds(0, ZCH)]

        pltpu.sync_copy(zero_r, z_v)
        for k in range(16):
            pltpu.sync_copy(z_v, agg_s.at[pl.ds(stripe + k * ZCH, ZCH)])
        plsc.subcore_barrier()

        def start_loads(u, b):
            base = (u * NS + sid) * SCH
            pltpu.async_copy(m_r.at[pl.ds(base, SCH)], m_v[b], sem[b])
            pltpu.async_copy(dst_r.at[pl.ds(base // 128, 1)], raw_v[b], sem[b])

        def finish_loads(u, b):
            base = (u * NS + sid) * SCH
            pltpu.make_async_copy(m_r.at[pl.ds(base, SCH)], m_v[b],
                                  sem[b]).wait()
            pltpu.make_async_copy(dst_r.at[pl.ds(base // 128, 1)], raw_v[b],
                                  sem[b]).wait()

        start_loads(0, 0)
        start_loads(1, 1)

        def pair_body(p, carry):
            for b in range(2):
                u = 2 * p + b
                finish_loads(u, b)
                for g in range(SCH // 16):
                    v = raw_v[b][0, pl.ds(g * 16, 16)]
                    lo = v - base_node
                    inb = (lo >= 0) & (lo < HALF)
                    idx_v[b][0, pl.ds(g * 16, 16)] = jnp.where(inb, lo, HALF)
                pltpu.sync_copy(m_v[b], agg_s.at[idx_v[b].at[0]], add=True)

                @pl.when(u + 2 < n_per_tile)
                def _():
                    start_loads(u + 2, b)
            return carry

        lax.fori_loop(0, n_pairs, pair_body, 0)
        plsc.subcore_barrier()
        for k in range(16):
            pltpu.sync_copy(agg_s.at[pl.ds(stripe + k * ZCH, ZCH)], z_v)
            pltpu.sync_copy(z_v, agg_r.at[pl.ds(base_node + stripe + k * ZCH, ZCH)])

    f = pl.kernel(
        body,
        out_type=jax.ShapeDtypeStruct((NP, D), F32),
        mesh=_mesh(),
        compiler_params=_sc_params(),
        scratch_types=[
            pltpu.VMEM((SCH, D), F32), pltpu.VMEM((SCH, D), F32),
            pltpu.VMEM((1, SCH), jnp.int32), pltpu.VMEM((1, SCH), jnp.int32),
            pltpu.VMEM((1, SCH), jnp.int32), pltpu.VMEM((1, SCH), jnp.int32),
            pltpu.SemaphoreType.DMA, pltpu.SemaphoreType.DMA,
            pltpu.VMEM_SHARED((HALF + 8, D), F32),
        ],
    )
    return f(m, dst2d, zeros)


def _bdot(x, w):
    return jnp.dot(x.astype(BF16), w.astype(BF16),
                   preferred_element_type=F32)


def _tc_edge_mlp(hs, hd, ea, w1sT, w1dT, w1eT, b1, w2T, b2):
    B = 1600
    grid = N_EDGES // B

    def body(hs_r, hd_r, ea_r, w1s_r, w1d_r, w1e_r, b1_r, w2_r, b2_r, o_r):
        t = _bdot(hs_r[...], w1s_r[...])
        t += _bdot(hd_r[...], w1d_r[...])
        t += _bdot(ea_r[...], w1e_r[...])
        t += b1_r[...]
        t = t * jax.nn.sigmoid(t)
        o_r[...] = _bdot(t, w2_r[...]) + b2_r[...]

    full = lambda shape: pl.BlockSpec(shape, lambda i: (0, 0))
    return pl.pallas_call(
        body,
        grid=(grid,),
        in_specs=[
            pl.BlockSpec((B, D), lambda i: (i, 0)),
            pl.BlockSpec((B, D), lambda i: (i, 0)),
            pl.BlockSpec((B, 4), lambda i: (i, 0)),
            full((D, D)), full((D, D)), full((4, D)), full((1, D)),
            full((D, D)), full((1, D)),
        ],
        out_specs=pl.BlockSpec((B, D), lambda i: (i, 0)),
        out_shape=jax.ShapeDtypeStruct((EP, D), F32),
    )(hs, hd, ea, w1sT, w1dT, w1eT, b1, w2T, b2)


def _tc_gru(h, agg, wih, bih, whh, bhh):
    B = 2048
    grid = NP // B

    def body(h_r, agg_r, wr_r, wz_r, wn_r, ur_r, uz_r, un_r,
             br_r, bz_r, bn_r, cr_r, cz_r, cn_r, o_r):
        h_v = h_r[...].astype(F32)
        a_v = agg_r[...]
        dot = lambda x, w: jnp.dot(x, w, preferred_element_type=F32,
                                   precision=HIGHEST)
        r = jax.nn.sigmoid(dot(a_v, wr_r[...]) + br_r[...]
                           + dot(h_v, ur_r[...]) + cr_r[...])
        z = jax.nn.sigmoid(dot(a_v, wz_r[...]) + bz_r[...]
                           + dot(h_v, uz_r[...]) + cz_r[...])
        nn_ = jnp.tanh(dot(a_v, wn_r[...]) + bn_r[...]
                       + r * (dot(h_v, un_r[...]) + cn_r[...]))
        o_r[...] = ((1.0 - z) * nn_ + z * h_v).astype(BF16)

    wihT = wih.T
    whhT = whh.T
    ws = [wihT[:, i * D:(i + 1) * D] for i in range(3)]
    us = [whhT[:, i * D:(i + 1) * D] for i in range(3)]
    bs = [bih.reshape(3, D)[i].reshape(1, D) for i in range(3)]
    cs = [bhh.reshape(3, D)[i].reshape(1, D) for i in range(3)]
    full = lambda shape: pl.BlockSpec(shape, lambda i: (0, 0))
    return pl.pallas_call(
        body,
        grid=(grid,),
        in_specs=[pl.BlockSpec((B, D), lambda i: (i, 0)),
                  pl.BlockSpec((B, D), lambda i: (i, 0))]
        + [full((D, D))] * 6 + [full((1, D))] * 6,
        out_specs=pl.BlockSpec((B, D), lambda i: (i, 0)),
        out_shape=jax.ShapeDtypeStruct((NP, D), BF16),
    )(h, agg, *ws, *us, *bs, *cs)


def _tc_head(hs, hd, ea, w1sT, w1dT, w1eT, b1, w2row, b2):
    B = 1600
    grid = N_EDGES // B

    def body(hs_r, hd_r, ea_r, w1s_r, w1d_r, w1e_r, b1_r, w2_r, b2_r, o_r):
        t = _bdot(hs_r[...], w1s_r[...])
        t += _bdot(hd_r[...], w1d_r[...])
        t += _bdot(ea_r[...], w1e_r[...])
        t += b1_r[...]
        t = t * jax.nn.sigmoid(t)
        lg = jnp.sum(t * w2_r[...], axis=1, keepdims=True) + b2_r[...]
        o_r[...] = jax.nn.sigmoid(lg)

    full = lambda shape: pl.BlockSpec(shape, lambda i: (0, 0))
    return pl.pallas_call(
        body,
        grid=(grid,),
        in_specs=[
            pl.BlockSpec((B, D), lambda i: (i, 0)),
            pl.BlockSpec((B, D), lambda i: (i, 0)),
            pl.BlockSpec((B, 4), lambda i: (i, 0)),
            full((D, D)), full((D, D)), full((4, D)), full((1, D)),
            full((1, D)), full((1, 1)),
        ],
        out_specs=pl.BlockSpec((B, 1), lambda i: (i, 0)),
        out_shape=jax.ShapeDtypeStruct((EP, 1), F32),
    )(hs, hd, ea, w1sT, w1dT, w1eT, b1, w2row, b2)


def kernel(edge_index, node_kind_ids, edge_attr, emb, msg_w1, msg_b1, msg_w2,
           msg_b2, gru_wih, gru_bih, gru_whh, gru_bhh, eh_w1, eh_b1, eh_w2,
           eh_b2):
    src = edge_index[0].astype(jnp.int32)
    dst = edge_index[1].astype(jnp.int32)
    src2 = jnp.pad(src, (0, EP - N_EDGES)).reshape(EP // 128, 128)
    dst2 = jnp.pad(dst, (0, EP - N_EDGES),
                   constant_values=1 << 28).reshape(EP // 128, 128)
    ids2 = jnp.pad(node_kind_ids.astype(jnp.int32),
                   (0, NP - N_NODES)).reshape(NP // 128, 128)
    zeros = jnp.zeros((ZCH, D), F32)

    (h,) = _sc_gather(emb.astype(BF16), (ids2,), 128)

    for l in range(3):
        w1 = msg_w1[l]
        w1sT = w1[:, :D].T
        w1dT = w1[:, D:2 * D].T
        w1eT = w1[:, 2 * D:].T
        b1 = msg_b1[l].reshape(1, D)
        w2T = msg_w2[l].T
        b2 = msg_b2[l].reshape(1, D)

        hs, hd = _sc_gather(h, (src2, dst2), GCH)
        m = _tc_edge_mlp(hs, hd, edge_attr, w1sT, w1dT, w1eT, b1, w2T, b2)
        agg = _sc_scatter_add(m, dst2, zeros)
        h = _tc_gru(h, agg, gru_wih[l], gru_bih[l], gru_whh[l], gru_bhh[l])

    hs, hd = _sc_gather(h, (src2, dst2), GCH)
    out = _tc_head(hs, hd, edge_attr, eh_w1[:, :D].T, eh_w1[:, D:2 * D].T,
                   eh_w1[:, 2 * D:].T, eh_b1.reshape(1, D),
                   eh_w2.reshape(1, D), eh_b2.reshape(1, 1))
    return out[:N_EDGES, 0]
